# baseline (speedup 1.0000x reference)
import jax
import jax.numpy as jnp
from jax.experimental import pallas as pl
from jax.experimental.pallas import tpu as pltpu


_CHUNKS = 4


def _up_kernel(x_ref, m_ref, o_ref, mb_ref):
    _, C, K = x_ref.shape
    _, _, Ho, Wo = o_ref.shape
    N = m_ref.shape[1]
    cn = N // _CHUNKS
    lo = Ho // _CHUNKS

    @pl.when(pl.program_id(0) == 0)
    def _():
        mb_ref[...] = m_ref[...].astype(jnp.bfloat16)

    xb = x_ref[0].astype(jnp.bfloat16)
    for t in range(_CHUNKS):
        rt = jnp.dot(xb, mb_ref[:, t * cn:(t + 1) * cn],
                     preferred_element_type=jnp.float32)
        o_ref[0, :, t * lo:(t + 1) * lo, :] = rt.reshape(C, lo, Wo)


def kernel(x, M):
    B, C, H, W = x.shape
    K = H * W
    N = M.shape[1]
    Ho = 2 * H
    Wo = N // Ho

    xf = x.reshape(B, C, K)
    y = pl.pallas_call(
        _up_kernel,
        out_shape=jax.ShapeDtypeStruct((B, C, Ho, Wo), jnp.float32),
        grid=(B,),
        in_specs=[
            pl.BlockSpec((1, C, K), lambda b: (b, 0, 0)),
            pl.BlockSpec((K, N), lambda b: (0, 0)),
        ],
        out_specs=pl.BlockSpec((1, C, Ho, Wo), lambda b: (b, 0, 0, 0)),
        scratch_shapes=[pltpu.VMEM((K, N), jnp.bfloat16)],
        compiler_params=pltpu.CompilerParams(
            dimension_semantics=("arbitrary",)),
    )(xf, M)
    return y


# speedup vs baseline: 2.5826x; 2.5826x over previous
import jax
import jax.numpy as jnp
from jax.experimental import pallas as pl
from jax.experimental.pallas import tpu as pltpu


_BB = 2


def _up_kernel(x_ref, m_ref, o_ref, mb_ref):
    BB, C, K = x_ref.shape
    N = m_ref.shape[1]

    @pl.when(pl.program_id(0) == 0)
    def _():
        mb_ref[...] = m_ref[...].astype(jnp.bfloat16)

    xb = x_ref[...].reshape(BB * C, K).astype(jnp.bfloat16)
    r = jnp.dot(xb, mb_ref[...], preferred_element_type=jnp.float32)
    o_ref[...] = r.reshape(BB, C, N)


def kernel(x, M):
    B, C, H, W = x.shape
    K = H * W
    N = M.shape[1]
    Ho = 2 * H
    Wo = N // Ho

    xf = x.reshape(B, C, K)
    yf = pl.pallas_call(
        _up_kernel,
        out_shape=jax.ShapeDtypeStruct((B, C, N), jnp.float32),
        grid=(B // _BB,),
        in_specs=[
            pl.BlockSpec((_BB, C, K), lambda i: (i, 0, 0)),
            pl.BlockSpec((K, N), lambda i: (0, 0)),
        ],
        out_specs=pl.BlockSpec((_BB, C, N), lambda i: (i, 0, 0)),
        scratch_shapes=[pltpu.VMEM((K, N), jnp.bfloat16)],
        compiler_params=pltpu.CompilerParams(
            dimension_semantics=("arbitrary",)),
    )(xf, M)
    return yf.reshape(B, C, Ho, Wo)
